# Initial kernel scaffold; baseline (speedup 1.0000x reference)
import jax
import jax.numpy as jnp
from jax.experimental import pallas as pl


def kernel(z_e, codebook):
    raise NotImplementedError("write your pallas kernel here")


# speedup vs baseline: 1.0265x; 1.0265x over previous
import functools

import jax
import jax.numpy as jnp
from jax import lax
from jax.experimental import pallas as pl
from jax.experimental.pallas import tpu as pltpu
from jax.experimental.pallas import tpu_sc as plsc

_K = 8192
_D = 4
_BETA = 0.25
_NTOK = 16384
_R = 256
_NW = 32
_BPW = _NTOK // _NW
_LANES = 16


def _tc_body(z_ref, cbt_ref, idx_ref, loss_ref, b2_ref):
    i = pl.program_id(0)

    @pl.when(i == 0)
    def _init():
        cbt = cbt_ref[...]
        r0 = cbt[0:1, :]
        r1 = cbt[1:2, :]
        r2 = cbt[2:3, :]
        r3 = cbt[3:4, :]
        b2_ref[...] = ((r0 * r0 + r1 * r1) + r2 * r2) + r3 * r3
        loss_ref[...] = jnp.zeros((1, 1), jnp.float32)

    z = z_ref[...]
    u = lax.bitcast_convert_type(z, jnp.uint32)
    u = u + jnp.uint32(0x7FFF) + ((u >> jnp.uint32(16)) & jnp.uint32(1))
    u = u & jnp.uint32(0xFFFF0000)
    zb = lax.bitcast_convert_type(u, jnp.float32)
    ab = lax.dot_general(zb, cbt_ref[...], (((1,), (0,)), ((), ())),
                         preferred_element_type=jnp.float32)
    z0 = z[:, 0:1]
    z1 = z[:, 1:2]
    z2 = z[:, 2:3]
    z3 = z[:, 3:4]
    a2 = ((z0 * z0 + z1 * z1) + z2 * z2) + z3 * z3
    s = a2 + b2_ref[...]
    d2 = s - ab * 2.0
    dist = jnp.sqrt(jnp.maximum(d2, 0.0))
    rowmin = jnp.min(dist, axis=1, keepdims=True)
    lane = lax.broadcasted_iota(jnp.int32, dist.shape, 1)
    cand = jnp.where(dist <= rowmin, lane, _K)
    idx_ref[0, 0, :] = jnp.min(cand, axis=1)

    part = jnp.sum(rowmin * rowmin)
    loss_ref[...] = loss_ref[...] + jnp.reshape(part, (1, 1))

    @pl.when(i == pl.num_programs(0) - 1)
    def _fin():
        loss_ref[...] = loss_ref[...] * ((1.0 + _BETA) / (_NTOK * _D))


def _tc_argmin(z_flat, cbt, interpret=False):
    return pl.pallas_call(
        _tc_body,
        grid=(_NTOK // _R,),
        in_specs=[
            pl.BlockSpec((_R, _D), lambda i: (i, 0)),
            pl.BlockSpec((_D, _K), lambda i: (0, 0)),
        ],
        out_specs=[
            pl.BlockSpec((1, 1, _R), lambda i: (i, 0, 0)),
            pl.BlockSpec((1, 1), lambda i: (0, 0)),
        ],
        out_shape=[
            jax.ShapeDtypeStruct((_NTOK // _R, 1, _R), jnp.int32),
            jax.ShapeDtypeStruct((1, 1), jnp.float32),
        ],
        scratch_shapes=[pltpu.VMEM((1, _K), jnp.float32)],
        interpret=interpret,
    )(z_flat, cbt)


@functools.cache
def _make_sc_gather():
    @functools.partial(
        pl.kernel,
        out_type=jax.ShapeDtypeStruct((_D, _NTOK), jnp.float32),
        mesh=plsc.VectorSubcoreMesh(
            core_axis_name="c", subcore_axis_name="s", num_cores=2, num_subcores=16),
        scratch_types=[
            pltpu.VMEM((_K * _D,), jnp.float32),
            pltpu.VMEM((_BPW,), jnp.int32),
            pltpu.VMEM((_D, _BPW), jnp.float32),
        ],
        compiler_params=pltpu.CompilerParams(needs_layout_passes=False),
    )
    def _sc_gather(cb_hbm, idx_hbm, out_hbm, cb_v, idx_v, out_v):
        wid = lax.axis_index("s") * 2 + lax.axis_index("c")
        base = wid * _BPW
        pltpu.sync_copy(cb_hbm, cb_v)
        pltpu.sync_copy(idx_hbm.at[pl.ds(base, _BPW)], idx_v)

        def step(i, carry):
            addr = idx_v[pl.ds(i * _LANES, _LANES)] * _D
            for d in range(_D):
                out_v[d, pl.ds(i * _LANES, _LANES)] = plsc.load_gather(cb_v, [addr + d])
            return carry

        lax.fori_loop(0, _BPW // _LANES, step, 0)
        for d in range(_D):
            pltpu.sync_copy(out_v.at[d], out_hbm.at[d, pl.ds(base, _BPW)])

    return _sc_gather


def kernel(z_e, codebook):
    B, C, H, W = z_e.shape
    z = jnp.transpose(z_e, (0, 2, 3, 1)).reshape(-1, _D)
    idx_blocks, loss = _tc_argmin(z, codebook.T)
    idx_flat = idx_blocks.reshape(_NTOK)
    zq_planes = _make_sc_gather()(codebook.reshape(-1), idx_flat)

    z_q = jnp.transpose(zq_planes.reshape(_D, B, H, W), (1, 0, 2, 3))
    indices_img = idx_flat.reshape(B, H, W)
    vq_loss = loss[0, 0]
    return (z_q, indices_img, z_q, vq_loss)
